# baseline (speedup 1.0000x reference)
import math

import jax
import jax.numpy as jnp
import numpy as np
from jax.experimental import pallas as pl
from jax.experimental.pallas import tpu as pltpu

_D = 768
_SEGS = ((0, 64), (64, 96), (96, 112))
_S_PAD = 128
_CHUNK = 512
_NBUF = 8
_SCALE = 1.0 / math.sqrt(_D)


def _attn_kernel(q_hbm, k0_ref, k1_ref, k2_ref, v0_ref, v1_ref, v2_ref,
                 s0_ref, s1_ref, s2_ref, m_ref, o_hbm,
                 kb_ref, vb_ref, b_ref, qbuf, obuf, insem, outsem):
    kb_ref[0:64, :] = (k0_ref[...] * _SCALE).astype(jnp.bfloat16)
    kb_ref[64:96, :] = (k1_ref[...] * _SCALE).astype(jnp.bfloat16)
    kb_ref[96:112, :] = (k2_ref[...] * _SCALE).astype(jnp.bfloat16)
    kb_ref[112:128, :] = jnp.zeros((16, _D), jnp.bfloat16)
    vb_ref[0:64, :] = (v0_ref[...] * (1.0 / 3.0)).astype(jnp.bfloat16)
    vb_ref[64:96, :] = (v1_ref[...] * (1.0 / 3.0)).astype(jnp.bfloat16)
    vb_ref[96:112, :] = (v2_ref[...] * (1.0 / 3.0)).astype(jnp.bfloat16)
    vb_ref[112:128, :] = jnp.zeros((16, _D), jnp.bfloat16)
    b_ref[0:1, 0:64] = s0_ref[...]
    b_ref[0:1, 64:96] = s1_ref[...]
    b_ref[0:1, 96:112] = s2_ref[...]
    b_ref[0:1, 112:128] = jnp.full((1, 16), -1e30, jnp.float32)

    nsteps = q_hbm.shape[0] // _CHUNK

    def in_copy(i, slot):
        return pltpu.make_async_copy(
            q_hbm.at[pl.ds(i * _CHUNK, _CHUNK), :], qbuf.at[slot],
            insem.at[slot])

    def out_copy(i, slot):
        return pltpu.make_async_copy(
            obuf.at[slot], o_hbm.at[pl.ds(i * _CHUNK, _CHUNK), :],
            outsem.at[slot])

    for i in range(_NBUF):
        in_copy(i, i).start()

    def step(i, carry):
        slot = jax.lax.rem(i, _NBUF)
        in_copy(i, slot).wait()

        @pl.when(i >= _NBUF)
        def _wait_prev_store():
            out_copy(i - _NBUF, slot).wait()

        q = qbuf[slot].astype(jnp.bfloat16)
        s = jax.lax.dot_general(
            q, kb_ref[...], (((1,), (1,)), ((), ())),
            preferred_element_type=jnp.float32,
        )
        s = s + b_ref[...]
        mx = jnp.max(s, axis=1, keepdims=True)
        e = jnp.exp(s - mx)
        denom = jax.lax.dot_general(
            e.astype(jnp.bfloat16), m_ref[...], (((1,), (0,)), ((), ())),
            preferred_element_type=jnp.float32,
        )
        p = (e / jnp.maximum(denom, 1e-30)).astype(jnp.bfloat16)
        obuf[slot] = jax.lax.dot_general(
            p, vb_ref[...], (((1,), (0,)), ((), ())),
            preferred_element_type=jnp.float32,
        )
        out_copy(i, slot).start()

        @pl.when(i + _NBUF < nsteps)
        def _prefetch():
            in_copy(i + _NBUF, slot).start()

        return carry

    jax.lax.fori_loop(0, nsteps, step, 0)
    for j in range(_NBUF):
        i = nsteps - _NBUF + j
        out_copy(i, i % _NBUF).wait()


def kernel(query, keys0, values0, salience0, keys1, values1, salience1,
           keys2, values2, salience2):
    B, T, D = query.shape
    n = B * T
    q = query.reshape(n, D)
    seg_of = np.full((_S_PAD,), -1, dtype=np.int32)
    for si, (lo, hi) in enumerate(_SEGS):
        seg_of[lo:hi] = si
    seg_mat = jnp.asarray(
        (seg_of[:, None] == seg_of[None, :]) & (seg_of[:, None] >= 0),
        dtype=jnp.bfloat16)

    vmem = pl.BlockSpec(memory_space=pltpu.MemorySpace.VMEM)
    out = pl.pallas_call(
        _attn_kernel,
        in_specs=[
            pl.BlockSpec(memory_space=pl.ANY),
            vmem, vmem, vmem, vmem, vmem, vmem, vmem, vmem, vmem, vmem,
        ],
        out_specs=pl.BlockSpec(memory_space=pl.ANY),
        out_shape=jax.ShapeDtypeStruct((n, D), jnp.float32),
        scratch_shapes=[
            pltpu.VMEM((_S_PAD, _D), jnp.bfloat16),
            pltpu.VMEM((_S_PAD, _D), jnp.bfloat16),
            pltpu.VMEM((1, _S_PAD), jnp.float32),
            pltpu.VMEM((_NBUF, _CHUNK, _D), jnp.float32),
            pltpu.VMEM((_NBUF, _CHUNK, _D), jnp.float32),
            pltpu.SemaphoreType.DMA((_NBUF,)),
            pltpu.SemaphoreType.DMA((_NBUF,)),
        ],
    )(q, keys0, keys1, keys2, values0, values1, values2,
      salience0.reshape(1, 64), salience1.reshape(1, 32),
      salience2.reshape(1, 16), seg_mat)
    return out.reshape(B, T, D)


# speedup vs baseline: 1.1229x; 1.1229x over previous
import math

import jax
import jax.numpy as jnp
import numpy as np
from jax.experimental import pallas as pl
from jax.experimental.pallas import tpu as pltpu

_D = 768
_SEGS = ((0, 64), (64, 96), (96, 112))
_S_PAD = 128
_CHUNK = 1024
_NBUF = 6
_SCALE = 1.0 / math.sqrt(_D)


def _attn_kernel(q_hbm, k0_ref, k1_ref, k2_ref, v0_ref, v1_ref, v2_ref,
                 s0_ref, s1_ref, s2_ref, m_ref, o_hbm,
                 kb_ref, vb_ref, b_ref, qbuf, obuf, insem, outsem):
    kb_ref[0:64, :] = (k0_ref[...] * _SCALE).astype(jnp.bfloat16)
    kb_ref[64:96, :] = (k1_ref[...] * _SCALE).astype(jnp.bfloat16)
    kb_ref[96:112, :] = (k2_ref[...] * _SCALE).astype(jnp.bfloat16)
    kb_ref[112:128, :] = jnp.zeros((16, _D), jnp.bfloat16)
    vb_ref[0:64, :] = (v0_ref[...] * (1.0 / 3.0)).astype(jnp.bfloat16)
    vb_ref[64:96, :] = (v1_ref[...] * (1.0 / 3.0)).astype(jnp.bfloat16)
    vb_ref[96:112, :] = (v2_ref[...] * (1.0 / 3.0)).astype(jnp.bfloat16)
    vb_ref[112:128, :] = jnp.zeros((16, _D), jnp.bfloat16)
    b_ref[0:1, 0:64] = s0_ref[...]
    b_ref[0:1, 64:96] = s1_ref[...]
    b_ref[0:1, 96:112] = s2_ref[...]
    b_ref[0:1, 112:128] = jnp.full((1, 16), -1e30, jnp.float32)

    nsteps = q_hbm.shape[0] // _CHUNK

    def in_copy(i, slot):
        return pltpu.make_async_copy(
            q_hbm.at[pl.ds(i * _CHUNK, _CHUNK), :], qbuf.at[slot],
            insem.at[slot])

    def out_copy(i, slot):
        return pltpu.make_async_copy(
            obuf.at[slot], o_hbm.at[pl.ds(i * _CHUNK, _CHUNK), :],
            outsem.at[slot])

    for i in range(_NBUF):
        in_copy(i, i).start()

    def step(i, carry):
        slot = jax.lax.rem(i, _NBUF)
        in_copy(i, slot).wait()

        @pl.when(i >= _NBUF)
        def _wait_prev_store():
            out_copy(i - _NBUF, slot).wait()

        q = qbuf[slot].astype(jnp.bfloat16)
        s = jax.lax.dot_general(
            q, kb_ref[...], (((1,), (1,)), ((), ())),
            preferred_element_type=jnp.float32,
        )
        s = s + b_ref[...]
        mx = jnp.max(s, axis=1, keepdims=True)
        e = jnp.exp(s - mx)
        denom = jax.lax.dot_general(
            e.astype(jnp.bfloat16), m_ref[...], (((1,), (0,)), ((), ())),
            preferred_element_type=jnp.float32,
        )
        p = (e / jnp.maximum(denom, 1e-30)).astype(jnp.bfloat16)
        obuf[slot] = jax.lax.dot_general(
            p, vb_ref[...], (((1,), (0,)), ((), ())),
            preferred_element_type=jnp.float32,
        )
        out_copy(i, slot).start()

        @pl.when(i + _NBUF < nsteps)
        def _prefetch():
            in_copy(i + _NBUF, slot).start()

        return carry

    jax.lax.fori_loop(0, nsteps, step, 0)
    for j in range(_NBUF):
        i = nsteps - _NBUF + j
        out_copy(i, i % _NBUF).wait()


def kernel(query, keys0, values0, salience0, keys1, values1, salience1,
           keys2, values2, salience2):
    B, T, D = query.shape
    n = B * T
    q = query.reshape(n, D)
    seg_of = np.full((_S_PAD,), -1, dtype=np.int32)
    for si, (lo, hi) in enumerate(_SEGS):
        seg_of[lo:hi] = si
    seg_mat = jnp.asarray(
        (seg_of[:, None] == seg_of[None, :]) & (seg_of[:, None] >= 0),
        dtype=jnp.bfloat16)

    vmem = pl.BlockSpec(memory_space=pltpu.MemorySpace.VMEM)
    out = pl.pallas_call(
        _attn_kernel,
        in_specs=[
            pl.BlockSpec(memory_space=pl.ANY),
            vmem, vmem, vmem, vmem, vmem, vmem, vmem, vmem, vmem, vmem,
        ],
        out_specs=pl.BlockSpec(memory_space=pl.ANY),
        out_shape=jax.ShapeDtypeStruct((n, D), jnp.float32),
        scratch_shapes=[
            pltpu.VMEM((_S_PAD, _D), jnp.bfloat16),
            pltpu.VMEM((_S_PAD, _D), jnp.bfloat16),
            pltpu.VMEM((1, _S_PAD), jnp.float32),
            pltpu.VMEM((_NBUF, _CHUNK, _D), jnp.float32),
            pltpu.VMEM((_NBUF, _CHUNK, _D), jnp.float32),
            pltpu.SemaphoreType.DMA((_NBUF,)),
            pltpu.SemaphoreType.DMA((_NBUF,)),
        ],
    )(q, keys0, keys1, keys2, values0, values1, values2,
      salience0.reshape(1, 64), salience1.reshape(1, 32),
      salience2.reshape(1, 16), seg_mat)
    return out.reshape(B, T, D)
